# Initial kernel scaffold; baseline (speedup 1.0000x reference)
import jax
import jax.numpy as jnp
from jax.experimental import pallas as pl
from jax.experimental.pallas import tpu as pltpu


def _make_body(coeffs, lanes):
    c0, c1, c2, c3 = coeffs

    def body(p0_ref, p1_ref, p2_ref, p3_ref, y_ref, o_ref):
        i = pl.program_id(1)

        @pl.when(i == 0)
        def _():
            o_ref[...] = jnp.zeros_like(o_ref)

        y = y_ref[...]
        d3 = p3_ref[...] - y
        G, H, W = d3.shape
        o_ref[...] += (c3 * (d3 * d3)).reshape(G * H // 8, 8, W).sum(axis=0)[None]

        d2 = p2_ref[...] - y[:, ::2, ::2]
        o_ref[0, :, : W // 2] += (
            (c2 * (d2 * d2)).reshape(G * H // 16, 8, W // 2).sum(axis=0)
        )
        d1 = p1_ref[...] - y[:, ::4, ::4]
        o_ref[0, :, : W // 4] += (
            (c1 * (d1 * d1)).reshape(G * H // 32, 8, W // 4).sum(axis=0)
        )
        d0 = p0_ref[...] - y[:, ::8, ::8]
        o_ref[0, :, : W // 8] += (
            (c0 * (d0 * d0)).reshape(G * H // 64, 8, W // 8).sum(axis=0)
        )

    return body


def kernel(y_pred0, y_pred1, y_pred2, y_pred3, y):
    B, C, H, W = y.shape
    N = B * C
    p0 = y_pred0.reshape(N, H // 8, W // 8)
    p1 = y_pred1.reshape(N, H // 4, W // 4)
    p2 = y_pred2.reshape(N, H // 2, W // 2)
    p3 = y_pred3.reshape(N, H, W)
    y3 = y.reshape(N, H, W)

    raw = [1.0 / 2**i for i in range(4)]
    wsum = sum(raw)
    wrev = [w / wsum for w in reversed(raw)]
    coeffs = tuple(
        wrev[l] / float(p.size)
        for l, p in enumerate((y_pred0, y_pred1, y_pred2, y_pred3))
    )

    G = 4
    nb = N // (2 * G)

    def im3(c, i):
        return (c * nb + i, 0, 0)

    out = pl.pallas_call(
        _make_body(coeffs, W),
        out_shape=jax.ShapeDtypeStruct((2, 8, W), jnp.float32),
        grid_spec=pltpu.PrefetchScalarGridSpec(
            num_scalar_prefetch=0,
            grid=(2, nb),
            in_specs=[
                pl.BlockSpec((G, H // 8, W // 8), im3),
                pl.BlockSpec((G, H // 4, W // 4), im3),
                pl.BlockSpec((G, H // 2, W // 2), im3),
                pl.BlockSpec((G, H, W), im3),
                pl.BlockSpec((G, H, W), im3),
            ],
            out_specs=pl.BlockSpec((1, 8, W), lambda c, i: (c, 0, 0)),
        ),
        compiler_params=pltpu.CompilerParams(
            dimension_semantics=("parallel", "arbitrary"),
            vmem_limit_bytes=64 * 1024 * 1024,
        ),
    )(p0, p1, p2, p3, y3)
    return jnp.sum(out)


# speedup vs baseline: 4.4746x; 4.4746x over previous
import jax
import jax.numpy as jnp
from jax import lax
from jax.experimental import pallas as pl
from jax.experimental.pallas import tpu as pltpu


def _downsample_lanes(y, stride, W):
    G, R, _ = y.shape
    win = min(128, W)
    h = win // stride
    idx = stride * lax.broadcasted_iota(jnp.int32, (G, R, h), 2)
    pieces = []
    for j in range(8 // stride):
        base = stride * j * W
        for m in range(W // win):
            chunk = y[:, :, base + win * m : base + win * (m + 1)]
            pieces.append(jnp.take_along_axis(chunk, idx, axis=2))
    return jnp.concatenate(pieces, axis=2)


def _make_body(coeffs, W):
    c0, c1, c2, c3 = coeffs

    def body(p0_ref, p1_ref, p2_ref, p3_ref, y_ref, o_ref):
        i = pl.program_id(1)

        @pl.when(i == 0)
        def _():
            o_ref[...] = jnp.zeros_like(o_ref)

        y = y_ref[...]
        d3 = p3_ref[...] - y
        o_ref[...] += (c3 * (d3 * d3)).sum(axis=0)[None]

        d2 = p2_ref[...] - _downsample_lanes(y, 2, W)
        o_ref[0, :, : 2 * W] += (c2 * (d2 * d2)).sum(axis=0)
        d1 = p1_ref[...] - _downsample_lanes(y, 4, W)
        o_ref[0, :, : W // 2] += (c1 * (d1 * d1)).sum(axis=0)
        d0 = p0_ref[...] - _downsample_lanes(y, 8, W)
        o_ref[0, :, : W // 8] += (c0 * (d0 * d0)).sum(axis=0)

    return body


def kernel(y_pred0, y_pred1, y_pred2, y_pred3, y):
    B, C, H, W = y.shape
    N = B * C
    R = H // 8
    p0 = y_pred0.reshape(N, R, W // 8)
    p1 = y_pred1.reshape(N, R, W // 2)
    p2 = y_pred2.reshape(N, R, 2 * W)
    p3 = y_pred3.reshape(N, R, 8 * W)
    y3 = y.reshape(N, R, 8 * W)

    raw = [1.0 / 2**i for i in range(4)]
    wsum = sum(raw)
    wrev = [w / wsum for w in reversed(raw)]
    coeffs = tuple(
        wrev[l] / float(p.size)
        for l, p in enumerate((y_pred0, y_pred1, y_pred2, y_pred3))
    )

    G = 4
    nb = N // (2 * G)

    def im3(c, i):
        return (c * nb + i, 0, 0)

    out = pl.pallas_call(
        _make_body(coeffs, W),
        out_shape=jax.ShapeDtypeStruct((2, R, 8 * W), jnp.float32),
        grid_spec=pltpu.PrefetchScalarGridSpec(
            num_scalar_prefetch=0,
            grid=(2, nb),
            in_specs=[
                pl.BlockSpec((G, R, W // 8), im3),
                pl.BlockSpec((G, R, W // 2), im3),
                pl.BlockSpec((G, R, 2 * W), im3),
                pl.BlockSpec((G, R, 8 * W), im3),
                pl.BlockSpec((G, R, 8 * W), im3),
            ],
            out_specs=pl.BlockSpec((1, R, 8 * W), lambda c, i: (c, 0, 0)),
        ),
        compiler_params=pltpu.CompilerParams(
            dimension_semantics=("parallel", "arbitrary"),
            vmem_limit_bytes=64 * 1024 * 1024,
        ),
    )(p0, p1, p2, p3, y3)
    return jnp.sum(out)


# speedup vs baseline: 4.7722x; 1.0665x over previous
import jax
import jax.numpy as jnp
from jax import lax
from jax.experimental import pallas as pl
from jax.experimental.pallas import tpu as pltpu


def _downsample_lanes(y, stride, W):
    G, R, _ = y.shape
    win = min(128, W)
    h = win // stride
    idx = stride * lax.broadcasted_iota(jnp.int32, (G, R, h), 2)
    pieces = []
    for j in range(8 // stride):
        base = stride * j * W
        for m in range(W // win):
            chunk = y[:, :, base + win * m : base + win * (m + 1)]
            pieces.append(jnp.take_along_axis(chunk, idx, axis=2))
    return jnp.concatenate(pieces, axis=2)


def _make_body(coeffs, W):
    c0, c1, c2, c3 = coeffs

    def body(p0_ref, p1_ref, p2_ref, p3_ref, y_ref, o_ref):
        i = pl.program_id(1)

        @pl.when(i == 0)
        def _():
            o_ref[...] = jnp.zeros_like(o_ref)

        y = y_ref[...]
        d3 = p3_ref[...] - y
        o_ref[...] += (c3 * (d3 * d3)).sum(axis=0)[None]

        d2 = p2_ref[...] - _downsample_lanes(y, 2, W)
        o_ref[0, :, : 2 * W] += (c2 * (d2 * d2)).sum(axis=0)
        d1 = p1_ref[...] - _downsample_lanes(y, 4, W)
        o_ref[0, :, : W // 2] += (c1 * (d1 * d1)).sum(axis=0)
        d0 = p0_ref[...] - _downsample_lanes(y, 8, W)
        o_ref[0, :, : W // 8] += (c0 * (d0 * d0)).sum(axis=0)

    return body


def kernel(y_pred0, y_pred1, y_pred2, y_pred3, y):
    B, C, H, W = y.shape
    N = B * C
    R = H // 8
    p0 = y_pred0.reshape(N, R, W // 8)
    p1 = y_pred1.reshape(N, R, W // 2)
    p2 = y_pred2.reshape(N, R, 2 * W)
    p3 = y_pred3.reshape(N, R, 8 * W)
    y3 = y.reshape(N, R, 8 * W)

    raw = [1.0 / 2**i for i in range(4)]
    wsum = sum(raw)
    wrev = [w / wsum for w in reversed(raw)]
    coeffs = tuple(
        wrev[l] / float(p.size)
        for l, p in enumerate((y_pred0, y_pred1, y_pred2, y_pred3))
    )

    G = 8
    nb = N // (2 * G)

    def im3(c, i):
        return (c * nb + i, 0, 0)

    out = pl.pallas_call(
        _make_body(coeffs, W),
        out_shape=jax.ShapeDtypeStruct((2, R, 8 * W), jnp.float32),
        grid_spec=pltpu.PrefetchScalarGridSpec(
            num_scalar_prefetch=0,
            grid=(2, nb),
            in_specs=[
                pl.BlockSpec((G, R, W // 8), im3),
                pl.BlockSpec((G, R, W // 2), im3),
                pl.BlockSpec((G, R, 2 * W), im3),
                pl.BlockSpec((G, R, 8 * W), im3),
                pl.BlockSpec((G, R, 8 * W), im3),
            ],
            out_specs=pl.BlockSpec((1, R, 8 * W), lambda c, i: (c, 0, 0)),
        ),
        compiler_params=pltpu.CompilerParams(
            dimension_semantics=("parallel", "arbitrary"),
            vmem_limit_bytes=64 * 1024 * 1024,
        ),
    )(p0, p1, p2, p3, y3)
    return jnp.sum(out)


# speedup vs baseline: 4.8192x; 1.0099x over previous
import jax
import jax.numpy as jnp
from jax import lax
from jax.experimental import pallas as pl
from jax.experimental.pallas import tpu as pltpu


def _downsample_lanes(y, stride, W):
    G, R, _ = y.shape
    win = min(128, W)
    h = win // stride
    idx = stride * lax.broadcasted_iota(jnp.int32, (G, R, h), 2)
    pieces = []
    for j in range(8 // stride):
        base = stride * j * W
        for m in range(W // win):
            chunk = y[:, :, base + win * m : base + win * (m + 1)]
            pieces.append(jnp.take_along_axis(chunk, idx, axis=2))
    return jnp.concatenate(pieces, axis=2)


def _make_body(coeffs, W):
    c0, c1, c2, c3 = coeffs

    def body(p0_ref, p1_ref, p2_ref, p3_ref, y_ref, o_ref):
        i = pl.program_id(1)

        @pl.when(i == 0)
        def _():
            o_ref[...] = jnp.zeros_like(o_ref)

        y = y_ref[...]
        d3 = p3_ref[...] - y
        o_ref[...] += (c3 * (d3 * d3)).sum(axis=0)[None]

        d2 = p2_ref[...] - _downsample_lanes(y, 2, W)
        o_ref[0, :, : 2 * W] += (c2 * (d2 * d2)).sum(axis=0)
        d1 = p1_ref[...] - _downsample_lanes(y, 4, W)
        o_ref[0, :, : W // 2] += (c1 * (d1 * d1)).sum(axis=0)
        d0 = p0_ref[...] - _downsample_lanes(y, 8, W)
        o_ref[0, :, : W // 8] += (c0 * (d0 * d0)).sum(axis=0)

    return body


def kernel(y_pred0, y_pred1, y_pred2, y_pred3, y):
    B, C, H, W = y.shape
    N = B * C
    R = H // 8
    p0 = y_pred0.reshape(N, R, W // 8)
    p1 = y_pred1.reshape(N, R, W // 2)
    p2 = y_pred2.reshape(N, R, 2 * W)
    p3 = y_pred3.reshape(N, R, 8 * W)
    y3 = y.reshape(N, R, 8 * W)

    raw = [1.0 / 2**i for i in range(4)]
    wsum = sum(raw)
    wrev = [w / wsum for w in reversed(raw)]
    coeffs = tuple(
        wrev[l] / float(p.size)
        for l, p in enumerate((y_pred0, y_pred1, y_pred2, y_pred3))
    )

    G = 16
    nb = N // (2 * G)

    def im3(c, i):
        return (c * nb + i, 0, 0)

    out = pl.pallas_call(
        _make_body(coeffs, W),
        out_shape=jax.ShapeDtypeStruct((2, R, 8 * W), jnp.float32),
        grid_spec=pltpu.PrefetchScalarGridSpec(
            num_scalar_prefetch=0,
            grid=(2, nb),
            in_specs=[
                pl.BlockSpec((G, R, W // 8), im3),
                pl.BlockSpec((G, R, W // 2), im3),
                pl.BlockSpec((G, R, 2 * W), im3),
                pl.BlockSpec((G, R, 8 * W), im3),
                pl.BlockSpec((G, R, 8 * W), im3),
            ],
            out_specs=pl.BlockSpec((1, R, 8 * W), lambda c, i: (c, 0, 0)),
        ),
        compiler_params=pltpu.CompilerParams(
            dimension_semantics=("parallel", "arbitrary"),
            vmem_limit_bytes=64 * 1024 * 1024,
        ),
    )(p0, p1, p2, p3, y3)
    return jnp.sum(out)


# speedup vs baseline: 4.8450x; 1.0053x over previous
import jax
import jax.numpy as jnp
from jax import lax
from jax.experimental import pallas as pl
from jax.experimental.pallas import tpu as pltpu


def _downsample_lanes(y, stride, W):
    G, R, _ = y.shape
    win = min(128, W)
    h = win // stride
    idx = stride * lax.broadcasted_iota(jnp.int32, (G, R, h), 2)
    pieces = []
    for j in range(8 // stride):
        base = stride * j * W
        for m in range(W // win):
            chunk = y[:, :, base + win * m : base + win * (m + 1)]
            pieces.append(jnp.take_along_axis(chunk, idx, axis=2))
    return jnp.concatenate(pieces, axis=2)


def _make_body(coeffs, W):
    c0, c1, c2, c3 = coeffs

    def body(p0_ref, p1_ref, p2_ref, p3_ref, y_ref, o_ref):
        i = pl.program_id(1)

        @pl.when(i == 0)
        def _():
            o_ref[...] = jnp.zeros_like(o_ref)

        y = y_ref[...]
        d3 = p3_ref[...] - y
        o_ref[...] += (c3 * (d3 * d3)).sum(axis=0)[None]

        d2 = p2_ref[...] - _downsample_lanes(y, 2, W)
        o_ref[0, :, : 2 * W] += (c2 * (d2 * d2)).sum(axis=0)
        d1 = p1_ref[...] - _downsample_lanes(y, 4, W)
        o_ref[0, :, : W // 2] += (c1 * (d1 * d1)).sum(axis=0)
        d0 = p0_ref[...] - _downsample_lanes(y, 8, W)
        o_ref[0, :, : W // 8] += (c0 * (d0 * d0)).sum(axis=0)

    return body


def kernel(y_pred0, y_pred1, y_pred2, y_pred3, y):
    B, C, H, W = y.shape
    N = B * C
    R = H // 8
    p0 = y_pred0.reshape(N, R, W // 8)
    p1 = y_pred1.reshape(N, R, W // 2)
    p2 = y_pred2.reshape(N, R, 2 * W)
    p3 = y_pred3.reshape(N, R, 8 * W)
    y3 = y.reshape(N, R, 8 * W)

    raw = [1.0 / 2**i for i in range(4)]
    wsum = sum(raw)
    wrev = [w / wsum for w in reversed(raw)]
    coeffs = tuple(
        wrev[l] / float(p.size)
        for l, p in enumerate((y_pred0, y_pred1, y_pred2, y_pred3))
    )

    G = 16
    nb = N // G

    def im3(c, i):
        return (c * nb + i, 0, 0)

    out = pl.pallas_call(
        _make_body(coeffs, W),
        out_shape=jax.ShapeDtypeStruct((2, R, 8 * W), jnp.float32),
        grid_spec=pltpu.PrefetchScalarGridSpec(
            num_scalar_prefetch=0,
            grid=(1, nb),
            in_specs=[
                pl.BlockSpec((G, R, W // 8), im3),
                pl.BlockSpec((G, R, W // 2), im3),
                pl.BlockSpec((G, R, 2 * W), im3),
                pl.BlockSpec((G, R, 8 * W), im3),
                pl.BlockSpec((G, R, 8 * W), im3),
            ],
            out_specs=pl.BlockSpec((1, R, 8 * W), lambda c, i: (c, 0, 0)),
        ),
        compiler_params=pltpu.CompilerParams(
            dimension_semantics=("parallel", "arbitrary"),
            vmem_limit_bytes=64 * 1024 * 1024,
        ),
    )(p0, p1, p2, p3, y3)
    return jnp.sum(out)


# speedup vs baseline: 4.9086x; 1.0131x over previous
import jax
import jax.numpy as jnp
from jax import lax
from jax.experimental import pallas as pl
from jax.experimental.pallas import tpu as pltpu


def _downsample_lanes(y, stride, W):
    G, R, _ = y.shape
    win = min(128, W)
    h = win // stride
    idx = stride * lax.broadcasted_iota(jnp.int32, (G, R, h), 2)
    pieces = []
    for j in range(8 // stride):
        base = stride * j * W
        for m in range(W // win):
            chunk = y[:, :, base + win * m : base + win * (m + 1)]
            pieces.append(jnp.take_along_axis(chunk, idx, axis=2))
    return jnp.concatenate(pieces, axis=2)


def _make_body(coeffs, W, nsteps):
    c0, c1, c2, c3 = coeffs

    def body(p0_ref, p1_ref, p2_ref, p3_ref, y_ref, o_ref, acc_ref):
        i = pl.program_id(0)

        @pl.when(i == 0)
        def _():
            acc_ref[...] = jnp.zeros_like(acc_ref)

        y = y_ref[...]
        d3 = p3_ref[...] - y
        acc_ref[...] += (c3 * (d3 * d3)).sum(axis=0)

        d2 = p2_ref[...] - _downsample_lanes(y, 2, W)
        acc_ref[:, : 2 * W] += (c2 * (d2 * d2)).sum(axis=0)
        d1 = p1_ref[...] - _downsample_lanes(y, 4, W)
        acc_ref[:, : W // 2] += (c1 * (d1 * d1)).sum(axis=0)
        d0 = p0_ref[...] - _downsample_lanes(y, 8, W)
        acc_ref[:, : W // 8] += (c0 * (d0 * d0)).sum(axis=0)

        @pl.when(i == nsteps - 1)
        def _():
            o_ref[0, 0] = jnp.sum(acc_ref[...])

    return body


def kernel(y_pred0, y_pred1, y_pred2, y_pred3, y):
    B, C, H, W = y.shape
    N = B * C
    R = H // 8
    p0 = y_pred0.reshape(N, R, W // 8)
    p1 = y_pred1.reshape(N, R, W // 2)
    p2 = y_pred2.reshape(N, R, 2 * W)
    p3 = y_pred3.reshape(N, R, 8 * W)
    y3 = y.reshape(N, R, 8 * W)

    raw = [1.0 / 2**i for i in range(4)]
    wsum = sum(raw)
    wrev = [w / wsum for w in reversed(raw)]
    coeffs = tuple(
        wrev[l] / float(p.size)
        for l, p in enumerate((y_pred0, y_pred1, y_pred2, y_pred3))
    )

    G = min(16, N)
    nb = N // G

    def im3(i):
        return (i, 0, 0)

    out = pl.pallas_call(
        _make_body(coeffs, W, nb),
        out_shape=jax.ShapeDtypeStruct((1, 1), jnp.float32),
        grid_spec=pltpu.PrefetchScalarGridSpec(
            num_scalar_prefetch=0,
            grid=(nb,),
            in_specs=[
                pl.BlockSpec((G, R, W // 8), im3),
                pl.BlockSpec((G, R, W // 2), im3),
                pl.BlockSpec((G, R, 2 * W), im3),
                pl.BlockSpec((G, R, 8 * W), im3),
                pl.BlockSpec((G, R, 8 * W), im3),
            ],
            out_specs=pl.BlockSpec(
                (1, 1), lambda i: (0, 0), memory_space=pltpu.MemorySpace.SMEM
            ),
            scratch_shapes=[pltpu.VMEM((R, 8 * W), jnp.float32)],
        ),
        compiler_params=pltpu.CompilerParams(
            dimension_semantics=("arbitrary",),
            vmem_limit_bytes=64 * 1024 * 1024,
        ),
    )(p0, p1, p2, p3, y3)
    return out[0, 0]
